# baseline (speedup 1.0000x reference)
import functools

import jax
import jax.numpy as jnp
from jax.experimental import pallas as pl
from jax.experimental.pallas import tpu as pltpu


def _body(mb_ref, nh_ref, nf_ref, w1a_t_ref, w1b_t_ref, w1c_t_ref,
          w2_t_ref, b1_ref, b2_ref, out_ref):
    mb = mb_ref[...]
    a = mb[:, :256] + mb[:, 256:]
    c = a[:, :128] + a[:, 128:]
    acc = jnp.dot(nh_ref[...], w1a_t_ref[...],
                  preferred_element_type=jnp.float32)
    acc += jnp.dot(c, w1c_t_ref[...], preferred_element_type=jnp.float32)
    acc += jnp.dot(nf_ref[...], w1b_t_ref[...],
                   preferred_element_type=jnp.float32)
    h = jnp.maximum(acc + b1_ref[...], 0.0)
    out = jnp.dot(h, w2_t_ref[...], preferred_element_type=jnp.float32)
    out_ref[...] = jnp.maximum(out + b2_ref[...], 0.0)


@functools.partial(jax.jit, static_argnames=("block_rows", "interpret"))
def _run(mb2, nh, nf, W1, b1, W2, b2, block_rows=1000, interpret=False):
    n, k_mb = mb2.shape
    nodemb = nh.shape[1]
    nfeat = nf.shape[1]
    hidden = W1.shape[0]
    out_dim = W2.shape[0]

    w1a_t = W1[:, :nodemb].T
    w1b_t = W1[:, nodemb:nodemb + nfeat].T
    w1c = W1[:, nodemb + nfeat:]
    w1c_t = jnp.tile(w1c, (1, 8)).T
    w2_t = W2.T
    b1r = b1.reshape(1, hidden)
    b2r = b2.reshape(1, out_dim)

    grid = n // block_rows
    return pl.pallas_call(
        _body,
        grid=(grid,),
        in_specs=[
            pl.BlockSpec((block_rows, k_mb), lambda i: (i, 0)),
            pl.BlockSpec((block_rows, nodemb), lambda i: (i, 0)),
            pl.BlockSpec((block_rows, nfeat), lambda i: (i, 0)),
            pl.BlockSpec((nodemb, hidden), lambda i: (0, 0)),
            pl.BlockSpec((nfeat, hidden), lambda i: (0, 0)),
            pl.BlockSpec((128, hidden), lambda i: (0, 0)),
            pl.BlockSpec((hidden, out_dim), lambda i: (0, 0)),
            pl.BlockSpec((1, hidden), lambda i: (0, 0)),
            pl.BlockSpec((1, out_dim), lambda i: (0, 0)),
        ],
        out_specs=pl.BlockSpec((block_rows, out_dim), lambda i: (i, 0)),
        out_shape=jax.ShapeDtypeStruct((n, out_dim), jnp.float32),
        compiler_params=pltpu.CompilerParams(
            dimension_semantics=("arbitrary",)),
        interpret=interpret,
    )(mb2, nh, nf, w1a_t, w1b_t, w1c_t, w2_t, b1r, b2r)


def kernel(mailbox_edge_hidden_representation, node_hidden_state,
           node_features, W1, b1, W2, b2):
    n, deg, emb = mailbox_edge_hidden_representation.shape
    mb2 = mailbox_edge_hidden_representation.reshape(n, deg * emb)
    return _run(mb2, node_hidden_state, node_features, W1, b1, W2, b2)
